# baseline (speedup 1.0000x reference)
import functools

import jax
import jax.numpy as jnp
from jax import lax
from jax.experimental import pallas as pl
from jax.experimental.pallas import tpu as pltpu
from jax.experimental.pallas import tpu_sc as plsc

D = 64
G = 128
SCALE = 8.0

_info = plsc.get_sparse_core_info()
NC, NS, L = _info.num_cores, _info.num_subcores, _info.num_lanes
NW = NC * NS


def _gather_scaled(emb, idx2d):
    ng_total = idx2d.shape[0]
    ng_per_w = ng_total // NW
    b_total = ng_total * G

    mesh = plsc.VectorSubcoreMesh(core_axis_name="c", subcore_axis_name="s")

    @functools.partial(
        pl.kernel,
        mesh=mesh,
        compiler_params=pltpu.CompilerParams(use_tc_tiling_on_sc=False),
        out_type=jax.ShapeDtypeStruct((b_total, D), jnp.float32),
        scratch_types=[
            pltpu.VMEM((ng_per_w, G), jnp.int32),
            pltpu.VMEM((G, D), jnp.float32),
            pltpu.SemaphoreType.DMA,
        ],
    )
    def k(emb_hbm, idx_hbm, out_hbm, idx_v, rows_v, sem):
        wid = lax.axis_index("s") * NC + lax.axis_index("c")
        g0 = wid * ng_per_w
        pltpu.sync_copy(idx_hbm.at[pl.ds(g0, ng_per_w)], idx_v)

        def group_body(g, _):
            pltpu.async_copy(emb_hbm.at[idx_v.at[g]], rows_v, sem).wait()

            def row_body(i, _):
                for j in range(D // L):
                    rows_v[i, pl.ds(j * L, L)] = rows_v[i, pl.ds(j * L, L)] * SCALE
                return 0

            lax.fori_loop(0, G, row_body, 0)
            pltpu.sync_copy(rows_v, out_hbm.at[pl.ds((g0 + g) * G, G)])
            return 0

        lax.fori_loop(0, ng_per_w, group_body, 0)

    return k(emb, idx2d)


def kernel(tokens, embedding):
    b, s = tokens.shape
    idx = tokens.astype(jnp.int32).reshape(b * s // G, G)
    out = _gather_scaled(embedding, idx)
    return out.reshape(b, s, D)


# speedup vs baseline: 1.2118x; 1.2118x over previous
import functools

import jax
import jax.numpy as jnp
from jax import lax
from jax.experimental import pallas as pl
from jax.experimental.pallas import tpu as pltpu
from jax.experimental.pallas import tpu_sc as plsc

D = 64
G = 128
SCALE = 8.0
NBUF = 4
RU = 4

_info = plsc.get_sparse_core_info()
NC, NS, L = _info.num_cores, _info.num_subcores, _info.num_lanes
NW = NC * NS


def _gather_scaled(emb, idx2d):
    ng_total = idx2d.shape[0]
    ng_per_w = ng_total // NW
    b_total = ng_total * G
    nt = ng_per_w // NBUF

    mesh = plsc.VectorSubcoreMesh(core_axis_name="c", subcore_axis_name="s")

    @functools.partial(
        pl.kernel,
        mesh=mesh,
        compiler_params=pltpu.CompilerParams(use_tc_tiling_on_sc=False),
        out_type=jax.ShapeDtypeStruct((b_total, D), jnp.float32),
        scratch_types=(
            [pltpu.VMEM((ng_per_w, G), jnp.int32)]
            + [pltpu.VMEM((G, D), jnp.float32) for _ in range(2 * NBUF)]
            + [pltpu.SemaphoreType.DMA for _ in range(2 * NBUF)]
        ),
    )
    def k(emb_hbm, idx_hbm, out_hbm, idx_v, *bufs_and_sems):
        a_bufs = bufs_and_sems[:NBUF]
        b_bufs = bufs_and_sems[NBUF:2 * NBUF]
        gsems = bufs_and_sems[2 * NBUF:3 * NBUF]
        ssems = bufs_and_sems[3 * NBUF:]

        wid = lax.axis_index("s") * NC + lax.axis_index("c")
        g0 = wid * ng_per_w
        pltpu.sync_copy(idx_hbm.at[pl.ds(g0, ng_per_w)], idx_v)

        def fire_gather(b, g):
            pltpu.async_copy(emb_hbm.at[idx_v.at[g]], a_bufs[b], gsems[b])

        def wait_gather(b, g):
            pltpu.make_async_copy(
                emb_hbm.at[idx_v.at[g]], a_bufs[b], gsems[b]).wait()

        def fire_store(b, g):
            pltpu.async_copy(
                b_bufs[b], out_hbm.at[pl.ds((g0 + g) * G, G)], ssems[b])

        def wait_store(b, g):
            pltpu.make_async_copy(
                b_bufs[b], out_hbm.at[pl.ds((g0 + g) * G, G)], ssems[b]).wait()

        def scale(b):
            src, dst = a_bufs[b], b_bufs[b]

            def rows(i, _):
                r0 = i * RU
                for rr in range(RU):
                    for j in range(D // L):
                        dst[r0 + rr, pl.ds(j * L, L)] = (
                            src[r0 + rr, pl.ds(j * L, L)] * SCALE)
                return 0

            lax.fori_loop(0, G // RU, rows, 0)

        for b in range(NBUF):
            fire_gather(b, b)

        for b in range(NBUF):
            wait_gather(b, b)
            scale(b)
            fire_gather(b, NBUF + b)
            fire_store(b, b)

        def step(t, _):
            for b in range(NBUF):
                g = t * NBUF + b
                wait_gather(b, g)
                wait_store(b, g - NBUF)
                scale(b)
                fire_gather(b, g + NBUF)
                fire_store(b, g)
            return 0

        lax.fori_loop(1, nt - 1, step, 0)

        for b in range(NBUF):
            g = (nt - 1) * NBUF + b
            wait_gather(b, g)
            wait_store(b, g - NBUF)
            scale(b)
            fire_store(b, g)

        for b in range(NBUF):
            wait_store(b, (nt - 1) * NBUF + b)

    return k(emb, idx2d)


def kernel(tokens, embedding):
    b, s = tokens.shape
    idx = tokens.astype(jnp.int32).reshape(b * s // G, G)
    out = _gather_scaled(embedding, idx)
    return out.reshape(b, s, D)
